# baseline (speedup 1.0000x reference)
import functools

import numpy as np
import jax
import jax.numpy as jnp
from jax import lax
from jax.experimental import pallas as pl
from jax.experimental.pallas import tpu as pltpu
from jax.experimental.pallas import tpu_sc as plsc

_H = 1024
_CROP = 768
_Y0 = 200
_SCALE = 1.25
_NTILES = 32
_RPT = _CROP // _NTILES
_STAGE = 21
_LANES = 16

_d = np.arange(_Y0, _Y0 + _CROP)
_u = _d % 10
_CSRC = (8 * (_d // 10) + _u - (_u > 3) - (_u > 7)).astype(np.int32)
_CMASK = ((_u != 3) & (_u != 7)).astype(np.float32)

_sp = (_d + 0.5) / _SCALE - 0.5
_i0 = np.floor(_sp).astype(np.int64)
_w1 = (_sp - _i0).astype(np.float32)
_WY = np.zeros((_CROP, _H), np.float32)
_WY[np.arange(_CROP), _i0] = np.float32(1.0) - _w1
_WY[np.arange(_CROP), _i0 + 1] = _w1

_WYT = np.ascontiguousarray(_WY.T)


def _sc_flow_valid(flow_f, valid_f, cidx, cmask):
    mesh = plsc.VectorSubcoreMesh(core_axis_name="c", subcore_axis_name="s")

    @functools.partial(
        pl.kernel,
        mesh=mesh,
        compiler_params=pltpu.CompilerParams(use_tc_tiling_on_sc=False,
                                             needs_layout_passes=False),
        cost_estimate=pl.CostEstimate(flops=20_000_000,
                                      bytes_accessed=20_000_000,
                                      transcendentals=0),
        out_type=[
            jax.ShapeDtypeStruct((2 * _CROP * _CROP,), jnp.float32),
            jax.ShapeDtypeStruct((_CROP * _CROP,), jnp.float32),
        ],
        scratch_types=[
            pltpu.VMEM((_STAGE * _H,), jnp.float32),
            pltpu.VMEM((_STAGE * _H,), jnp.float32),
            pltpu.VMEM((_STAGE * _H,), jnp.int32),
            pltpu.VMEM((_CROP,), jnp.int32),
            pltpu.VMEM((_CROP,), jnp.float32),
            pltpu.VMEM((_RPT * _CROP,), jnp.float32),
            pltpu.VMEM((_RPT * _CROP,), jnp.float32),
            pltpu.VMEM((_RPT * _CROP,), jnp.float32),
        ],
    )
    def k(flow_hbm, valid_hbm, cidx_hbm, cmask_hbm, fout_hbm, vout_hbm,
          buf0, buf1, bufv, cidx_v, cmask_v, out0, out1, outv):
        wid = lax.axis_index("s") * 2 + lax.axis_index("c")
        base = wid * _RPT

        pltpu.sync_copy(cidx_hbm, cidx_v)
        pltpu.sync_copy(cmask_hbm, cmask_v)

        db = base + _Y0
        ub = db % 10
        smin = (8 * (db // 10) + ub - (ub > 3).astype(jnp.int32)
                - (ub > 7).astype(jnp.int32))
        pltpu.sync_copy(flow_hbm.at[pl.ds(smin * _H, _STAGE * _H)], buf0)
        pltpu.sync_copy(flow_hbm.at[pl.ds(_H * _H + smin * _H,
                                          _STAGE * _H)], buf1)
        pltpu.sync_copy(valid_hbm.at[pl.ds(smin * _H, _STAGE * _H)], bufv)

        @plsc.parallel_loop(0, _RPT)
        def row_body(r):
            d = base + r + _Y0
            u = d % 10
            soff = (8 * (d // 10) + u - (u > 3).astype(jnp.int32)
                    - (u > 7).astype(jnp.int32)) - smin
            rmask = jnp.where((u != 3) & (u != 7), jnp.float32(1.0),
                              jnp.float32(0.0))
            fbase = jnp.broadcast_to(soff * _H, (_LANES,))
            rmv = jnp.broadcast_to(rmask, (_LANES,))
            obase = r * _CROP

            for c in range(_CROP // _LANES):
                cs = pl.ds(c * _LANES, _LANES)
                ci = cidx_v[cs]
                cm = cmask_v[cs]
                fidx = fbase + ci
                g0 = plsc.load_gather(buf0, [fidx])
                g1 = plsc.load_gather(buf1, [fidx])
                gv = plsc.load_gather(bufv, [fidx])
                mf = jnp.where(gv != 0, rmv, jnp.float32(0.0)) * cm
                os_ = pl.ds(obase + c * _LANES, _LANES)
                out0[os_] = (g0 * jnp.float32(_SCALE)) * mf
                out1[os_] = (g1 * jnp.float32(_SCALE)) * mf
                outv[os_] = mf

        nout = _RPT * _CROP
        pltpu.sync_copy(out0, fout_hbm.at[pl.ds(base * _CROP, nout)])
        pltpu.sync_copy(out1, fout_hbm.at[pl.ds(_CROP * _CROP + base * _CROP,
                                                nout)])
        pltpu.sync_copy(outv, vout_hbm.at[pl.ds(base * _CROP, nout)])

    return k(flow_f, valid_f, cidx, cmask)


def _tc_resize(img1, img2, wy, wyt):
    def body(a_ref, b_ref, wy_ref, wyt_ref, o1_ref, o2_ref):
        wyb = wy_ref[...].astype(jnp.bfloat16)
        wytb = wyt_ref[...].astype(jnp.bfloat16)
        a = a_ref[0].astype(jnp.bfloat16)
        b = b_ref[0].astype(jnp.bfloat16)
        t1 = jnp.dot(wyb, a, preferred_element_type=jnp.float32)
        t2 = jnp.dot(wyb, b, preferred_element_type=jnp.float32)
        o1_ref[0] = jnp.dot(t1.astype(jnp.bfloat16), wytb,
                            preferred_element_type=jnp.float32)
        o2_ref[0] = jnp.dot(t2.astype(jnp.bfloat16), wytb,
                            preferred_element_type=jnp.float32)

    return pl.pallas_call(
        body,
        grid=(3,),
        in_specs=[
            pl.BlockSpec((1, _H, _H), lambda c: (c, 0, 0)),
            pl.BlockSpec((1, _H, _H), lambda c: (c, 0, 0)),
            pl.BlockSpec((_CROP, _H), lambda c: (0, 0)),
            pl.BlockSpec((_H, _CROP), lambda c: (0, 0)),
        ],
        out_specs=[
            pl.BlockSpec((1, _CROP, _CROP), lambda c: (c, 0, 0)),
            pl.BlockSpec((1, _CROP, _CROP), lambda c: (c, 0, 0)),
        ],
        out_shape=[
            jax.ShapeDtypeStruct((3, _CROP, _CROP), jnp.float32),
            jax.ShapeDtypeStruct((3, _CROP, _CROP), jnp.float32),
        ],
    )(img1, img2, wy, wyt)


def kernel(img1, img2, flow, valid):
    flow_f = flow.reshape(2 * _H * _H)
    valid_f = valid.reshape(_H * _H)
    fout_f, vout_f = _sc_flow_valid(flow_f, valid_f, _CSRC, _CMASK)
    img1c, img2c = _tc_resize(img1, img2, _WY, _WYT)
    flowc = fout_f.reshape(2, _CROP, _CROP)
    validc = vout_f.reshape(_CROP, _CROP)
    return img1c, img2c, flowc, validc


# speedup vs baseline: 1.1943x; 1.1943x over previous
import functools

import numpy as np
import jax
import jax.numpy as jnp
from jax import lax
from jax.experimental import pallas as pl
from jax.experimental.pallas import tpu as pltpu
from jax.experimental.pallas import tpu_sc as plsc

_H = 1024
_CROP = 768
_Y0 = 200
_SCALE = 1.25
_NTILES = 32
_RPT = _CROP // _NTILES
_STAGE = 21
_LANES = 16

_d = np.arange(_Y0, _Y0 + _CROP)
_u = _d % 10
_CSRC = (8 * (_d // 10) + _u - (_u > 3) - (_u > 7)).astype(np.int32)
_CMASK = ((_u != 3) & (_u != 7)).astype(np.float32)
_CSRC_ADJ = _CSRC - np.int32(128)

_sp = (_d + 0.5) / _SCALE - 0.5
_i0 = np.floor(_sp).astype(np.int64)
_w1 = (_sp - _i0).astype(np.float32)
_WY = np.zeros((_CROP, _H), np.float32)
_WY[np.arange(_CROP), _i0] = np.float32(1.0) - _w1
_WY[np.arange(_CROP), _i0 + 1] = _w1

_WYT = np.ascontiguousarray(_WY.T)


def _sc_flow_valid(flow, valid, cidx, cmask):
    mesh = plsc.VectorSubcoreMesh(core_axis_name="c", subcore_axis_name="s")

    @functools.partial(
        pl.kernel,
        mesh=mesh,
        compiler_params=pltpu.CompilerParams(use_tc_tiling_on_sc=True,
                                             needs_layout_passes=False),
        cost_estimate=pl.CostEstimate(flops=20_000_000,
                                      bytes_accessed=20_000_000,
                                      transcendentals=0),
        out_type=[
            jax.ShapeDtypeStruct((2, _CROP, _CROP), jnp.float32),
            jax.ShapeDtypeStruct((_CROP, _CROP), jnp.float32),
        ],
        scratch_types=[
            pltpu.VMEM((32, _CROP), jnp.float32),
            pltpu.VMEM((32, _CROP), jnp.float32),
            pltpu.VMEM((32, _CROP), jnp.int32),
            pltpu.VMEM((_CROP,), jnp.int32),
            pltpu.VMEM((_CROP,), jnp.float32),
            pltpu.VMEM((_RPT, _CROP), jnp.float32),
            pltpu.VMEM((_RPT, _CROP), jnp.float32),
            pltpu.VMEM((_RPT, _CROP), jnp.float32),
        ],
    )
    def k(flow_hbm, valid_hbm, cidx_hbm, cmask_hbm, fout_hbm, vout_hbm,
          buf0, buf1, bufv, cidx_v, cmask_v, out0, out1, outv):
        wid = lax.axis_index("s") * 2 + lax.axis_index("c")
        base = wid * _RPT

        pltpu.sync_copy(cidx_hbm, cidx_v)
        pltpu.sync_copy(cmask_hbm, cmask_v)

        db = base + _Y0
        ub = db % 10
        smin = (8 * (db // 10) + ub - (ub > 3).astype(jnp.int32)
                - (ub > 7).astype(jnp.int32))
        rowbase = (smin // 8) * 8
        pltpu.sync_copy(flow_hbm.at[0, pl.ds(rowbase, 32), pl.ds(128, _CROP)],
                        buf0)
        pltpu.sync_copy(flow_hbm.at[1, pl.ds(rowbase, 32), pl.ds(128, _CROP)],
                        buf1)
        pltpu.sync_copy(valid_hbm.at[pl.ds(rowbase, 32), pl.ds(128, _CROP)],
                        bufv)

        @plsc.parallel_loop(0, _RPT)
        def row_body(r):
            d = base + r + _Y0
            u = d % 10
            soff = (8 * (d // 10) + u - (u > 3).astype(jnp.int32)
                    - (u > 7).astype(jnp.int32)) - rowbase
            rmask = jnp.where((u != 3) & (u != 7), jnp.float32(1.0),
                              jnp.float32(0.0))
            offv = jnp.broadcast_to(soff, (_LANES,))
            rmv = jnp.broadcast_to(rmask, (_LANES,))

            for c in range(_CROP // _LANES):
                cs = pl.ds(c * _LANES, _LANES)
                ci = cidx_v[cs]
                cm = cmask_v[cs]
                g0 = plsc.load_gather(buf0, [offv, ci])
                g1 = plsc.load_gather(buf1, [offv, ci])
                gv = plsc.load_gather(bufv, [offv, ci])
                mf = jnp.where(gv != 0, rmv, jnp.float32(0.0)) * cm
                out0[r, cs] = (g0 * jnp.float32(_SCALE)) * mf
                out1[r, cs] = (g1 * jnp.float32(_SCALE)) * mf
                outv[r, cs] = mf

        pltpu.sync_copy(out0, fout_hbm.at[0, pl.ds(base, _RPT)])
        pltpu.sync_copy(out1, fout_hbm.at[1, pl.ds(base, _RPT)])
        pltpu.sync_copy(outv, vout_hbm.at[pl.ds(base, _RPT)])

    return k(flow, valid, cidx, cmask)


def _tc_resize(img1, img2, wy, wyt):
    def body(a_ref, b_ref, wy_ref, wyt_ref, o1_ref, o2_ref):
        wyb = wy_ref[...].astype(jnp.bfloat16)
        wytb = wyt_ref[...].astype(jnp.bfloat16)
        a = a_ref[0].astype(jnp.bfloat16)
        b = b_ref[0].astype(jnp.bfloat16)
        t1 = jnp.dot(wyb, a, preferred_element_type=jnp.float32)
        t2 = jnp.dot(wyb, b, preferred_element_type=jnp.float32)
        o1_ref[0] = jnp.dot(t1.astype(jnp.bfloat16), wytb,
                            preferred_element_type=jnp.float32)
        o2_ref[0] = jnp.dot(t2.astype(jnp.bfloat16), wytb,
                            preferred_element_type=jnp.float32)

    return pl.pallas_call(
        body,
        grid=(3,),
        in_specs=[
            pl.BlockSpec((1, _H, _H), lambda c: (c, 0, 0)),
            pl.BlockSpec((1, _H, _H), lambda c: (c, 0, 0)),
            pl.BlockSpec((_CROP, _H), lambda c: (0, 0)),
            pl.BlockSpec((_H, _CROP), lambda c: (0, 0)),
        ],
        out_specs=[
            pl.BlockSpec((1, _CROP, _CROP), lambda c: (c, 0, 0)),
            pl.BlockSpec((1, _CROP, _CROP), lambda c: (c, 0, 0)),
        ],
        out_shape=[
            jax.ShapeDtypeStruct((3, _CROP, _CROP), jnp.float32),
            jax.ShapeDtypeStruct((3, _CROP, _CROP), jnp.float32),
        ],
    )(img1, img2, wy, wyt)


def kernel(img1, img2, flow, valid):
    flowc, validc = _sc_flow_valid(flow, valid, _CSRC_ADJ, _CMASK)
    img1c, img2c = _tc_resize(img1, img2, _WY, _WYT)
    return img1c, img2c, flowc, validc


# speedup vs baseline: 1.4265x; 1.1943x over previous
import functools

import numpy as np
import jax
import jax.numpy as jnp
from jax import lax
from jax.experimental import pallas as pl
from jax.experimental.pallas import tpu as pltpu
from jax.experimental.pallas import tpu_sc as plsc

_H = 1024
_CROP = 768
_Y0 = 200
_SCALE = 1.25
_NTILES = 32
_RPT = _CROP // _NTILES
_STAGE = 21
_LANES = 16

_d = np.arange(_Y0, _Y0 + _CROP)
_u = _d % 10
_CSRC = (8 * (_d // 10) + _u - (_u > 3) - (_u > 7)).astype(np.int32)
_CMASK = ((_u != 3) & (_u != 7)).astype(np.float32)
_CSRC_ADJ = _CSRC - np.int32(128)

_sp = (_d + 0.5) / _SCALE - 0.5
_i0 = np.floor(_sp).astype(np.int64)
_w1 = (_sp - _i0).astype(np.float32)
_WY = np.zeros((_CROP, _H), np.float32)
_WY[np.arange(_CROP), _i0] = np.float32(1.0) - _w1
_WY[np.arange(_CROP), _i0 + 1] = _w1

_WYT = np.ascontiguousarray(_WY.T)


def _sc_flow_valid(flow, valid, cidx, cmask):
    mesh = plsc.VectorSubcoreMesh(core_axis_name="c", subcore_axis_name="s")

    @functools.partial(
        pl.kernel,
        mesh=mesh,
        compiler_params=pltpu.CompilerParams(use_tc_tiling_on_sc=True,
                                             needs_layout_passes=False),
        cost_estimate=pl.CostEstimate(flops=20_000_000,
                                      bytes_accessed=20_000_000,
                                      transcendentals=0),
        out_type=[
            jax.ShapeDtypeStruct((2, _CROP, _CROP), jnp.float32),
            jax.ShapeDtypeStruct((_CROP, _CROP), jnp.float32),
        ],
        scratch_types=[
            pltpu.VMEM((32, _CROP), jnp.float32),
            pltpu.VMEM((32, _CROP), jnp.float32),
            pltpu.VMEM((32, _CROP), jnp.int32),
            pltpu.VMEM((_CROP,), jnp.int32),
            pltpu.VMEM((_CROP,), jnp.float32),
            pltpu.VMEM((_RPT, _CROP), jnp.float32),
            pltpu.VMEM((_RPT, _CROP), jnp.float32),
            pltpu.VMEM((_RPT, _CROP), jnp.float32),
        ],
    )
    def k(flow_hbm, valid_hbm, cidx_hbm, cmask_hbm, fout_hbm, vout_hbm,
          buf0, buf1, bufv, cidx_v, cmask_v, out0, out1, outv):
        wid = lax.axis_index("s") * 2 + lax.axis_index("c")
        base = wid * _RPT

        pltpu.sync_copy(cidx_hbm, cidx_v)
        pltpu.sync_copy(cmask_hbm, cmask_v)

        db = base + _Y0
        ub = db % 10
        smin = (8 * (db // 10) + ub - (ub > 3).astype(jnp.int32)
                - (ub > 7).astype(jnp.int32))
        rowbase = (smin // 8) * 8
        pltpu.sync_copy(flow_hbm.at[0, pl.ds(rowbase, 32), pl.ds(128, _CROP)],
                        buf0)
        pltpu.sync_copy(flow_hbm.at[1, pl.ds(rowbase, 32), pl.ds(128, _CROP)],
                        buf1)
        pltpu.sync_copy(valid_hbm.at[pl.ds(rowbase, 32), pl.ds(128, _CROP)],
                        bufv)

        @plsc.parallel_loop(0, _RPT)
        def row_body(r):
            d = base + r + _Y0
            u = d % 10
            soff = (8 * (d // 10) + u - (u > 3).astype(jnp.int32)
                    - (u > 7).astype(jnp.int32)) - rowbase
            rmask = jnp.where((u != 3) & (u != 7), jnp.float32(1.0),
                              jnp.float32(0.0))
            offv = jnp.broadcast_to(soff, (_LANES,))
            rmv = jnp.broadcast_to(rmask, (_LANES,))

            @plsc.parallel_loop(0, _CROP, step=_LANES, unroll=4)
            def col_body(cc):
                cs = pl.ds(cc, _LANES)
                ci = cidx_v[cs]
                cm = cmask_v[cs]
                g0 = plsc.load_gather(buf0, [offv, ci])
                g1 = plsc.load_gather(buf1, [offv, ci])
                gv = plsc.load_gather(bufv, [offv, ci])
                mf = jnp.where(gv != 0, rmv, jnp.float32(0.0)) * cm
                out0[r, cs] = (g0 * jnp.float32(_SCALE)) * mf
                out1[r, cs] = (g1 * jnp.float32(_SCALE)) * mf
                outv[r, cs] = mf

        pltpu.sync_copy(out0, fout_hbm.at[0, pl.ds(base, _RPT)])
        pltpu.sync_copy(out1, fout_hbm.at[1, pl.ds(base, _RPT)])
        pltpu.sync_copy(outv, vout_hbm.at[pl.ds(base, _RPT)])

    return k(flow, valid, cidx, cmask)


def _tc_resize(img1, img2, wy, wyt):
    def body(a_ref, b_ref, wy_ref, wyt_ref, o1_ref, o2_ref):
        wyb = wy_ref[...].astype(jnp.bfloat16)
        wytb = wyt_ref[...].astype(jnp.bfloat16)
        a = a_ref[0].astype(jnp.bfloat16)
        b = b_ref[0].astype(jnp.bfloat16)
        t1 = jnp.dot(wyb, a, preferred_element_type=jnp.float32)
        t2 = jnp.dot(wyb, b, preferred_element_type=jnp.float32)
        o1_ref[0] = jnp.dot(t1.astype(jnp.bfloat16), wytb,
                            preferred_element_type=jnp.float32)
        o2_ref[0] = jnp.dot(t2.astype(jnp.bfloat16), wytb,
                            preferred_element_type=jnp.float32)

    return pl.pallas_call(
        body,
        grid=(3,),
        in_specs=[
            pl.BlockSpec((1, _H, _H), lambda c: (c, 0, 0)),
            pl.BlockSpec((1, _H, _H), lambda c: (c, 0, 0)),
            pl.BlockSpec((_CROP, _H), lambda c: (0, 0)),
            pl.BlockSpec((_H, _CROP), lambda c: (0, 0)),
        ],
        out_specs=[
            pl.BlockSpec((1, _CROP, _CROP), lambda c: (c, 0, 0)),
            pl.BlockSpec((1, _CROP, _CROP), lambda c: (c, 0, 0)),
        ],
        out_shape=[
            jax.ShapeDtypeStruct((3, _CROP, _CROP), jnp.float32),
            jax.ShapeDtypeStruct((3, _CROP, _CROP), jnp.float32),
        ],
    )(img1, img2, wy, wyt)


def kernel(img1, img2, flow, valid):
    flowc, validc = _sc_flow_valid(flow, valid, _CSRC_ADJ, _CMASK)
    img1c, img2c = _tc_resize(img1, img2, _WY, _WYT)
    return img1c, img2c, flowc, validc
